# Initial kernel scaffold; baseline (speedup 1.0000x reference)
import jax
import jax.numpy as jnp
from jax.experimental import pallas as pl


def kernel(source_ufea, target_ufea, source_user_idx, source_item_idx, source_vals, target_user_idx, target_item_idx, target_vals, W_gc1, b_gc1, W_gc2, b_gc2, W_gc3m, b_gc3m, W_gc3s, b_gc3s, W_gc4m, b_gc4m, W_gc4s, b_gc4s, W_sum, b_sum, W_sls, b_sls, W_tum, b_tum, W_tls, b_tls):
    raise NotImplementedError("write your pallas kernel here")


# speedup vs baseline: 1.1912x; 1.1912x over previous
import functools
import math

import jax
import jax.numpy as jnp
from jax import lax
from jax.experimental import pallas as pl
from jax.experimental.pallas import tpu as pltpu
from jax.experimental.pallas import tpu_sc as plsc

_D = 128
_ALPHA = 0.1
_E = 480000
_NT = 16
_NC = 2
_EB = 128
_EPT = _E // _NT
_EPT_PAD = 30720
_NB = _EPT_PAD // _EB
_SCB = 48
_NSC = _NB // _SCB
_CH = 64

_N_SRC_U = 29999
_N_TGT_U = 29999
_N_ITEM = 20000
_ITEM_PAD = 20480
_USERS_PAD = 30720
_OVERLAP = 10000
_TOTAL_USERS = 50000
_BR = 1024


def _leaky(x):
    return jnp.where(x > 0, x, _ALPHA * x)


def _make_spmm(n_rows_pad, n_passes):
    seg = n_rows_pad // (_NC * n_passes)
    stripe = seg // _NT
    ch = next(c for c in (64, 48, 32, 16, 8) if stripe % c == 0)
    nchunks = stripe // ch

    def body(x_hbm, cols_hbm, rows_hbm, vals_hbm, out_hbm,
             acc, cols_v, rows_v, vals_v, idxb0, idxb1, g0, g1,
             sem_g0, sem_g1, sem_s0, sem_s1):
        c = lax.axis_index("c")
        s = lax.axis_index("s")

        def one_pass(p, pcarry):
            base = (p * _NC + c) * seg

            def zfill(r, carry):
                for q in range(_D // 16):
                    g0[r, pl.ds(q * 16, 16)] = jnp.zeros((16,), jnp.float32)
                return carry

            lax.fori_loop(0, ch, zfill, 0)
            zslice = g0.at[pl.ds(0, ch)]

            def zinit(k, carry):
                off = pl.multiple_of(s * stripe + k * ch, ch)
                pltpu.sync_copy(zslice, acc.at[pl.ds(off, ch)])
                return carry

            lax.fori_loop(0, nchunks, zinit, 0)
            plsc.subcore_barrier()

            def process(b, g, idxb, s_off):
                for k in range(_EB // 32):
                    vv = vals_v[b, pl.ds(s_off + k * 16, 16)]
                    rr = rows_v[b, pl.ds(s_off + k * 16, 16)]
                    local = rr - base
                    valid = jnp.logical_and(local >= 0, local < seg)
                    idxb[pl.ds(k * 16, 16)] = jnp.where(valid, local, seg)
                    for j in range(16):
                        v = vv[j]
                        e = k * 16 + j
                        for q in range(_D // 16):
                            sl = pl.ds(q * 16, 16)
                            g[e, sl] = g[e, sl] * v

            def batch(b, carry):
                d0 = pltpu.async_copy(
                    x_hbm.at[cols_v.at[b, pl.ds(0, _EB // 2)]], g0, sem_g0)
                d1 = pltpu.async_copy(
                    x_hbm.at[cols_v.at[b, pl.ds(_EB // 2, _EB // 2)]],
                    g1, sem_g1)
                d0.wait()
                process(b, g0, idxb0, 0)
                s0 = pltpu.async_copy(g0, acc.at[idxb0], sem_s0, add=True)
                d1.wait()
                process(b, g1, idxb1, _EB // 2)
                s1 = pltpu.async_copy(g1, acc.at[idxb1], sem_s1, add=True)
                s0.wait()
                s1.wait()
                return carry

            def chunk(ci, carry):
                off = pl.multiple_of(ci * _SCB, _SCB)
                pltpu.sync_copy(cols_hbm.at[s, pl.ds(off, _SCB)], cols_v)
                pltpu.sync_copy(rows_hbm.at[s, pl.ds(off, _SCB)], rows_v)
                pltpu.sync_copy(vals_hbm.at[s, pl.ds(off, _SCB)], vals_v)
                lax.fori_loop(0, _SCB, batch, 0)
                return carry

            lax.fori_loop(0, _NSC, chunk, 0)
            plsc.subcore_barrier()

            def readout(k, carry):
                off = pl.multiple_of(s * stripe + k * ch, ch)
                pltpu.sync_copy(acc.at[pl.ds(off, ch)], g0.at[pl.ds(0, ch)])
                pltpu.sync_copy(g0.at[pl.ds(0, ch)],
                                out_hbm.at[pl.ds(base + off, ch)])
                return carry

            lax.fori_loop(0, nchunks, readout, 0)
            plsc.subcore_barrier()
            return pcarry

        lax.fori_loop(0, n_passes, one_pass, 0)

    return pl.kernel(
        body,
        out_type=jax.ShapeDtypeStruct((n_rows_pad, _D), jnp.float32),
        mesh=plsc.VectorSubcoreMesh(core_axis_name="c", subcore_axis_name="s",
                                    num_cores=_NC, num_subcores=_NT),
        scratch_types=[
            pltpu.VMEM_SHARED((seg + 8, _D), jnp.float32),
            pltpu.VMEM((_SCB, _EB), jnp.int32),
            pltpu.VMEM((_SCB, _EB), jnp.int32),
            pltpu.VMEM((_SCB, _EB), jnp.float32),
            pltpu.VMEM((_EB // 2,), jnp.int32),
            pltpu.VMEM((_EB // 2,), jnp.int32),
            pltpu.VMEM((_EB // 2, _D), jnp.float32),
            pltpu.VMEM((_EB // 2, _D), jnp.float32),
            pltpu.SemaphoreType.DMA,
            pltpu.SemaphoreType.DMA,
            pltpu.SemaphoreType.DMA,
            pltpu.SemaphoreType.DMA,
        ],
    )


def _hidden_body(x_ref, w_ref, b_ref, o_ref):
    o_ref[...] = _leaky(
        jnp.dot(x_ref[...], w_ref[...], preferred_element_type=jnp.float32)
        + b_ref[0])


def _hidden(x, w, b):
    n = x.shape[0]
    row_spec = pl.BlockSpec((_BR, _D), lambda i: (i, 0))
    return pl.pallas_call(
        _hidden_body,
        grid=(n // _BR,),
        in_specs=[
            row_spec,
            pl.BlockSpec((_D, _D), lambda i: (0, 0)),
            pl.BlockSpec((1, _D), lambda i: (0, 0)),
        ],
        out_specs=row_spec,
        out_shape=jax.ShapeDtypeStruct((n, _D), jnp.float32),
    )(x, w, b.reshape(1, _D))


_LOG_S2 = 0.1 + 0.9 * math.log(2.0)
_INV2S2 = 1.0 / (2.0 * math.exp(2.0 * _LOG_S2))


def _final_body(a2s, a2t, su, tu,
                w3m, w3s, w4m, w4s,
                wsum0, wsum1, wsls0, wsls1, wtum0, wtum1, wtls0, wtls1,
                b3m, b3s, b4m, b4s, bsum, bsls, btum, btls,
                a_ref, b_ref, c_ref, kld_ref):
    i = pl.program_id(0)

    def gcn(a2, w, bias):
        return _leaky(
            jnp.dot(a2[...], w[...], preferred_element_type=jnp.float32)
            + bias[0])

    s_mean = gcn(a2s, w3m, b3m)
    s_ls = gcn(a2s, w3s, b3s)
    t_mean = gcn(a2t, w4m, b4m)
    t_ls = gcn(a2t, w4s, b4s)
    suf = su[...]
    tuf = tu[...]

    def proj(h, x, w0, w1, bias):
        return (jnp.dot(h, w0[...], preferred_element_type=jnp.float32)
                + jnp.dot(x, w1[...], preferred_element_type=jnp.float32)
                + bias[0])

    sm2 = proj(s_mean, suf, wsum0, wsum1, bsum)
    sl2 = proj(s_ls, suf, wsls0, wsls1, bsls)
    tm2 = proj(t_mean, tuf, wtum0, wtum1, btum)
    tl2 = proj(t_ls, tuf, wtls0, wtls1, btls)

    a_ref[...] = 0.5 * (sm2 + tm2)
    b_ref[...] = 0.5 * tm2
    c_ref[...] = 0.5 * sm2

    rows = i * _BR + lax.broadcasted_iota(jnp.int32, (_BR, _D), 0)
    joint = rows < (_OVERLAP - 1)
    tail = jnp.logical_and(rows >= (_OVERLAP - 1), rows < _N_SRC_U)

    def klsum(mu, ls, mask):
        l1 = 0.1 + 0.9 * jnp.logaddexp(ls, 0.0)
        kl = (_LOG_S2 - l1) + (jnp.exp(2.0 * l1) + mu * mu) * _INV2S2 - 0.5
        return jnp.sum(jnp.where(mask, kl, 0.0))

    tot = (klsum(0.5 * (sm2 + tm2), 0.5 * (sl2 + tl2), joint)
           + klsum(0.5 * tm2, 0.5 * tl2, tail)
           + klsum(0.5 * sm2, 0.5 * sl2, tail))

    @pl.when(i == 0)
    def _():
        kld_ref[...] = jnp.zeros((1, 1), jnp.float32)

    kld_ref[...] = kld_ref[...] + jnp.reshape(tot, (1, 1))


def _final(a2s, a2t, su_p, tu_p, weights, biases):
    n = _USERS_PAD
    row_spec = pl.BlockSpec((_BR, _D), lambda i: (i, 0))
    full_w = pl.BlockSpec((_D, _D), lambda i: (0, 0))
    full_b = pl.BlockSpec((1, _D), lambda i: (0, 0))
    return pl.pallas_call(
        _final_body,
        grid=(n // _BR,),
        in_specs=(
            [row_spec] * 4
            + [full_w] * 12
            + [full_b] * 8
        ),
        out_specs=[row_spec, row_spec, row_spec,
                   pl.BlockSpec((1, 1), lambda i: (0, 0))],
        out_shape=[
            jax.ShapeDtypeStruct((n, _D), jnp.float32),
            jax.ShapeDtypeStruct((n, _D), jnp.float32),
            jax.ShapeDtypeStruct((n, _D), jnp.float32),
            jax.ShapeDtypeStruct((1, 1), jnp.float32),
        ],
    )(a2s, a2t, su_p, tu_p, *weights, *[b.reshape(1, _D) for b in biases])


def kernel(source_ufea, target_ufea, source_user_idx, source_item_idx,
           source_vals, target_user_idx, target_item_idx, target_vals,
           W_gc1, b_gc1, W_gc2, b_gc2, W_gc3m, b_gc3m, W_gc3s, b_gc3s,
           W_gc4m, b_gc4m, W_gc4s, b_gc4s, W_sum, b_sum, W_sls, b_sls,
           W_tum, b_tum, W_tls, b_tls):
    def esh(a):
        a2 = a.reshape(_NT, _EPT)
        a2 = jnp.pad(a2, ((0, 0), (0, _EPT_PAD - _EPT)))
        return a2.reshape(_NT, _NB, _EB)

    su_i = esh(source_user_idx.astype(jnp.int32))
    si_i = esh(source_item_idx.astype(jnp.int32))
    sv = esh(source_vals)
    tu_i = esh(target_user_idx.astype(jnp.int32))
    ti_i = esh(target_item_idx.astype(jnp.int32))
    tv = esh(target_vals)

    spmm_item = _make_spmm(_ITEM_PAD, 1)
    spmm_user = _make_spmm(_USERS_PAD, 2)

    agg1_s = spmm_item(source_ufea, su_i, si_i, sv)
    agg1_t = spmm_item(target_ufea, tu_i, ti_i, tv)
    hs = _hidden(agg1_s, W_gc1, b_gc1)
    ht = _hidden(agg1_t, W_gc2, b_gc2)
    agg2_s = spmm_user(hs, si_i, su_i, sv)
    agg2_t = spmm_user(ht, ti_i, tu_i, tv)

    su_p = jnp.pad(source_ufea, ((0, _USERS_PAD - _N_SRC_U), (0, 0)))
    tu_p = jnp.pad(target_ufea, ((0, _USERS_PAD - _N_TGT_U), (0, 0)))
    weights = [W_gc3m, W_gc3s, W_gc4m, W_gc4s,
               W_sum[:_D], W_sum[_D:], W_sls[:_D], W_sls[_D:],
               W_tum[:_D], W_tum[_D:], W_tls[:_D], W_tls[_D:]]
    biases = [b_gc3m, b_gc3s, b_gc4m, b_gc4s, b_sum, b_sls, b_tum, b_tls]
    a_arr, b_arr, c_arr, kld_arr = _final(agg2_s, agg2_t, su_p, tu_p,
                                          weights, biases)

    user = jnp.concatenate([
        jnp.zeros((1, _D), jnp.float32),
        a_arr[: _OVERLAP - 1],
        b_arr[_OVERLAP - 1: _N_TGT_U],
        c_arr[_OVERLAP - 1: _N_SRC_U],
    ], axis=0)
    kld = kld_arr[0, 0] / jnp.float32(_TOTAL_USERS)
    return user, kld


# speedup vs baseline: 1.2179x; 1.0224x over previous
import functools
import math

import jax
import jax.numpy as jnp
from jax import lax
from jax.experimental import pallas as pl
from jax.experimental.pallas import tpu as pltpu
from jax.experimental.pallas import tpu_sc as plsc

_D = 128
_ALPHA = 0.1
_E = 480000
_NT = 16
_NC = 2
_EB = 128
_EPT = _E // _NT
_EPT_PAD = 30720
_NB = _EPT_PAD // _EB
_SCB = 48
_NSC = _NB // _SCB
_CH = 64

_N_SRC_U = 29999
_N_TGT_U = 29999
_N_ITEM = 20000
_ITEM_PAD = 20480
_USERS_PAD = 30720
_OVERLAP = 10000
_TOTAL_USERS = 50000
_BR = 1024


def _leaky(x):
    return jnp.where(x > 0, x, _ALPHA * x)


def _make_spmm(n_rows_pad, n_passes):
    seg = n_rows_pad // (_NC * n_passes)
    stripe = seg // _NT
    ch = next(c for c in (64, 48, 32, 16, 8) if stripe % c == 0)
    nchunks = stripe // ch

    def body(x_hbm, cols_hbm, rows_hbm, vals_hbm, out_hbm,
             acc, cols_v, rows_v, vals_v, idxb0, idxb1, g0, g1,
             sem_g0, sem_g1, sem_s0, sem_s1):
        c = lax.axis_index("c")
        s = lax.axis_index("s")

        def one_pass(p, pcarry):
            base = (p * _NC + c) * seg

            def zfill(r, carry):
                for q in range(_D // 16):
                    g0[r, pl.ds(q * 16, 16)] = jnp.zeros((16,), jnp.float32)
                return carry

            lax.fori_loop(0, ch, zfill, 0)
            zslice = g0.at[pl.ds(0, ch)]

            def zinit(k, carry):
                off = pl.multiple_of(s * stripe + k * ch, ch)
                pltpu.sync_copy(zslice, acc.at[pl.ds(off, ch)])
                return carry

            lax.fori_loop(0, nchunks, zinit, 0)
            plsc.subcore_barrier()

            lanes0 = lax.iota(jnp.int32, 16)

            def process(b, g, idxb, s_off):
                for k in range(_EB // 32):
                    vv = vals_v[b, pl.ds(s_off + k * 16, 16)]
                    rr = rows_v[b, pl.ds(s_off + k * 16, 16)]
                    local = rr - base
                    valid = jnp.logical_and(local >= 0, local < seg)
                    idxb[pl.ds(k * 16, 16)] = jnp.where(
                        valid, local, seg + (lanes0 & 7))
                    for j in range(16):
                        v = vv[j]
                        e = k * 16 + j
                        for q in range(_D // 16):
                            sl = pl.ds(q * 16, 16)
                            g[e, sl] = g[e, sl] * v

            def batch(b, carry):
                d0 = pltpu.async_copy(
                    x_hbm.at[cols_v.at[b, pl.ds(0, _EB // 2)]], g0, sem_g0)
                d1 = pltpu.async_copy(
                    x_hbm.at[cols_v.at[b, pl.ds(_EB // 2, _EB // 2)]],
                    g1, sem_g1)
                d0.wait()
                process(b, g0, idxb0, 0)
                s0 = pltpu.async_copy(g0, acc.at[idxb0], sem_s0, add=True)
                d1.wait()
                process(b, g1, idxb1, _EB // 2)
                s1 = pltpu.async_copy(g1, acc.at[idxb1], sem_s1, add=True)
                s0.wait()
                s1.wait()
                return carry

            def chunk(ci, carry):
                off = pl.multiple_of(ci * _SCB, _SCB)
                pltpu.sync_copy(cols_hbm.at[s, pl.ds(off, _SCB)], cols_v)
                pltpu.sync_copy(rows_hbm.at[s, pl.ds(off, _SCB)], rows_v)
                pltpu.sync_copy(vals_hbm.at[s, pl.ds(off, _SCB)], vals_v)
                lax.fori_loop(0, _SCB, batch, 0)
                return carry

            lax.fori_loop(0, _NSC, chunk, 0)
            plsc.subcore_barrier()

            def readout(k, carry):
                off = pl.multiple_of(s * stripe + k * ch, ch)
                pltpu.sync_copy(acc.at[pl.ds(off, ch)], g0.at[pl.ds(0, ch)])
                pltpu.sync_copy(g0.at[pl.ds(0, ch)],
                                out_hbm.at[pl.ds(base + off, ch)])
                return carry

            lax.fori_loop(0, nchunks, readout, 0)
            plsc.subcore_barrier()
            return pcarry

        lax.fori_loop(0, n_passes, one_pass, 0)

    return pl.kernel(
        body,
        out_type=jax.ShapeDtypeStruct((n_rows_pad, _D), jnp.float32),
        mesh=plsc.VectorSubcoreMesh(core_axis_name="c", subcore_axis_name="s",
                                    num_cores=_NC, num_subcores=_NT),
        scratch_types=[
            pltpu.VMEM_SHARED((seg + 8, _D), jnp.float32),
            pltpu.VMEM((_SCB, _EB), jnp.int32),
            pltpu.VMEM((_SCB, _EB), jnp.int32),
            pltpu.VMEM((_SCB, _EB), jnp.float32),
            pltpu.VMEM((_EB // 2,), jnp.int32),
            pltpu.VMEM((_EB // 2,), jnp.int32),
            pltpu.VMEM((_EB // 2, _D), jnp.float32),
            pltpu.VMEM((_EB // 2, _D), jnp.float32),
            pltpu.SemaphoreType.DMA,
            pltpu.SemaphoreType.DMA,
            pltpu.SemaphoreType.DMA,
            pltpu.SemaphoreType.DMA,
        ],
    )


def _hidden_body(x_ref, w_ref, b_ref, o_ref):
    o_ref[...] = _leaky(
        jnp.dot(x_ref[...], w_ref[...], preferred_element_type=jnp.float32)
        + b_ref[0])


def _hidden(x, w, b):
    n = x.shape[0]
    row_spec = pl.BlockSpec((_BR, _D), lambda i: (i, 0))
    return pl.pallas_call(
        _hidden_body,
        grid=(n // _BR,),
        in_specs=[
            row_spec,
            pl.BlockSpec((_D, _D), lambda i: (0, 0)),
            pl.BlockSpec((1, _D), lambda i: (0, 0)),
        ],
        out_specs=row_spec,
        out_shape=jax.ShapeDtypeStruct((n, _D), jnp.float32),
    )(x, w, b.reshape(1, _D))


_LOG_S2 = 0.1 + 0.9 * math.log(2.0)
_INV2S2 = 1.0 / (2.0 * math.exp(2.0 * _LOG_S2))


def _final_body(a2s, a2t, su, tu,
                w3m, w3s, w4m, w4s,
                wsum0, wsum1, wsls0, wsls1, wtum0, wtum1, wtls0, wtls1,
                b3m, b3s, b4m, b4s, bsum, bsls, btum, btls,
                a_ref, b_ref, c_ref, kld_ref):
    i = pl.program_id(0)

    def gcn(a2, w, bias):
        return _leaky(
            jnp.dot(a2[...], w[...], preferred_element_type=jnp.float32)
            + bias[0])

    s_mean = gcn(a2s, w3m, b3m)
    s_ls = gcn(a2s, w3s, b3s)
    t_mean = gcn(a2t, w4m, b4m)
    t_ls = gcn(a2t, w4s, b4s)
    suf = su[...]
    tuf = tu[...]

    def proj(h, x, w0, w1, bias):
        return (jnp.dot(h, w0[...], preferred_element_type=jnp.float32)
                + jnp.dot(x, w1[...], preferred_element_type=jnp.float32)
                + bias[0])

    sm2 = proj(s_mean, suf, wsum0, wsum1, bsum)
    sl2 = proj(s_ls, suf, wsls0, wsls1, bsls)
    tm2 = proj(t_mean, tuf, wtum0, wtum1, btum)
    tl2 = proj(t_ls, tuf, wtls0, wtls1, btls)

    a_ref[...] = 0.5 * (sm2 + tm2)
    b_ref[...] = 0.5 * tm2
    c_ref[...] = 0.5 * sm2

    rows = i * _BR + lax.broadcasted_iota(jnp.int32, (_BR, _D), 0)
    joint = rows < (_OVERLAP - 1)
    tail = jnp.logical_and(rows >= (_OVERLAP - 1), rows < _N_SRC_U)

    def klsum(mu, ls, mask):
        l1 = 0.1 + 0.9 * jnp.logaddexp(ls, 0.0)
        kl = (_LOG_S2 - l1) + (jnp.exp(2.0 * l1) + mu * mu) * _INV2S2 - 0.5
        return jnp.sum(jnp.where(mask, kl, 0.0))

    tot = (klsum(0.5 * (sm2 + tm2), 0.5 * (sl2 + tl2), joint)
           + klsum(0.5 * tm2, 0.5 * tl2, tail)
           + klsum(0.5 * sm2, 0.5 * sl2, tail))

    @pl.when(i == 0)
    def _():
        kld_ref[...] = jnp.zeros((1, 1), jnp.float32)

    kld_ref[...] = kld_ref[...] + jnp.reshape(tot, (1, 1))


def _final(a2s, a2t, su_p, tu_p, weights, biases):
    n = _USERS_PAD
    row_spec = pl.BlockSpec((_BR, _D), lambda i: (i, 0))
    full_w = pl.BlockSpec((_D, _D), lambda i: (0, 0))
    full_b = pl.BlockSpec((1, _D), lambda i: (0, 0))
    return pl.pallas_call(
        _final_body,
        grid=(n // _BR,),
        in_specs=(
            [row_spec] * 4
            + [full_w] * 12
            + [full_b] * 8
        ),
        out_specs=[row_spec, row_spec, row_spec,
                   pl.BlockSpec((1, 1), lambda i: (0, 0))],
        out_shape=[
            jax.ShapeDtypeStruct((n, _D), jnp.float32),
            jax.ShapeDtypeStruct((n, _D), jnp.float32),
            jax.ShapeDtypeStruct((n, _D), jnp.float32),
            jax.ShapeDtypeStruct((1, 1), jnp.float32),
        ],
    )(a2s, a2t, su_p, tu_p, *weights, *[b.reshape(1, _D) for b in biases])


def kernel(source_ufea, target_ufea, source_user_idx, source_item_idx,
           source_vals, target_user_idx, target_item_idx, target_vals,
           W_gc1, b_gc1, W_gc2, b_gc2, W_gc3m, b_gc3m, W_gc3s, b_gc3s,
           W_gc4m, b_gc4m, W_gc4s, b_gc4s, W_sum, b_sum, W_sls, b_sls,
           W_tum, b_tum, W_tls, b_tls):
    def esh(a):
        a2 = a.reshape(_NT, _EPT)
        a2 = jnp.pad(a2, ((0, 0), (0, _EPT_PAD - _EPT)))
        return a2.reshape(_NT, _NB, _EB)

    su_i = esh(source_user_idx.astype(jnp.int32))
    si_i = esh(source_item_idx.astype(jnp.int32))
    sv = esh(source_vals)
    tu_i = esh(target_user_idx.astype(jnp.int32))
    ti_i = esh(target_item_idx.astype(jnp.int32))
    tv = esh(target_vals)

    spmm_item = _make_spmm(_ITEM_PAD, 1)
    spmm_user = _make_spmm(_USERS_PAD, 2)

    agg1_s = spmm_item(source_ufea, su_i, si_i, sv)
    agg1_t = spmm_item(target_ufea, tu_i, ti_i, tv)
    hs = _hidden(agg1_s, W_gc1, b_gc1)
    ht = _hidden(agg1_t, W_gc2, b_gc2)
    agg2_s = spmm_user(hs, si_i, su_i, sv)
    agg2_t = spmm_user(ht, ti_i, tu_i, tv)

    su_p = jnp.pad(source_ufea, ((0, _USERS_PAD - _N_SRC_U), (0, 0)))
    tu_p = jnp.pad(target_ufea, ((0, _USERS_PAD - _N_TGT_U), (0, 0)))
    weights = [W_gc3m, W_gc3s, W_gc4m, W_gc4s,
               W_sum[:_D], W_sum[_D:], W_sls[:_D], W_sls[_D:],
               W_tum[:_D], W_tum[_D:], W_tls[:_D], W_tls[_D:]]
    biases = [b_gc3m, b_gc3s, b_gc4m, b_gc4s, b_sum, b_sls, b_tum, b_tls]
    a_arr, b_arr, c_arr, kld_arr = _final(agg2_s, agg2_t, su_p, tu_p,
                                          weights, biases)

    user = jnp.concatenate([
        jnp.zeros((1, _D), jnp.float32),
        a_arr[: _OVERLAP - 1],
        b_arr[_OVERLAP - 1: _N_TGT_U],
        c_arr[_OVERLAP - 1: _N_SRC_U],
    ], axis=0)
    kld = kld_arr[0, 0] / jnp.float32(_TOTAL_USERS)
    return user, kld
